# baseline (speedup 1.0000x reference)
import functools

import jax
import jax.numpy as jnp
from jax import lax
from jax.experimental import pallas as pl
from jax.experimental.pallas import tpu as pltpu
from jax.experimental.pallas import tpu_sc as plsc

ROWS = 16384
COLS = 26
DIM = 16
FIELD = 100000
B = ROWS * COLS
NC = 2
NS = 16
NW = NC * NS
BPW = B // NW
CHUNK = 1024
NCHUNK = BPW // CHUNK
LANES = 16


def _embed_body(x_hbm, table_hbm, out_hbm, idx_v, rows0, rows1, gsem0, gsem1):
    wid = lax.axis_index("s") * NC + lax.axis_index("c")
    base = wid * BPW

    pltpu.sync_copy(x_hbm.at[pl.ds(base, BPW)], idx_v)

    lane = lax.iota(jnp.int32, LANES)
    def add_off(i, carry):
        off = pl.multiple_of(i * LANES, LANES)
        col = lax.rem(off + lane, COLS)
        idx_v[pl.ds(off, LANES)] = idx_v[pl.ds(off, LANES)] + col * FIELD
        return carry
    lax.fori_loop(0, BPW // LANES, add_off, 0)

    bufs = (rows0, rows1)
    sems = (gsem0, gsem1)

    def fire(j):
        return pltpu.async_copy(
            table_hbm.at[idx_v.at[pl.ds(j * CHUNK, CHUNK)]],
            bufs[j % 2], sems[j % 2])

    prev = fire(0)
    for j in range(1, NCHUNK):
        cur = fire(j)
        prev.wait()
        pltpu.sync_copy(bufs[(j - 1) % 2],
                        out_hbm.at[pl.ds(base + (j - 1) * CHUNK, CHUNK)])
        prev = cur
    prev.wait()
    pltpu.sync_copy(bufs[(NCHUNK - 1) % 2],
                    out_hbm.at[pl.ds(base + (NCHUNK - 1) * CHUNK, CHUNK)])


_embed_call = pl.kernel(
    _embed_body,
    out_type=jax.ShapeDtypeStruct((B, DIM), jnp.float32),
    mesh=plsc.VectorSubcoreMesh(core_axis_name="c", subcore_axis_name="s"),
    compiler_params=pltpu.CompilerParams(use_tc_tiling_on_sc=False),
    scratch_types=[
        pltpu.VMEM((BPW,), jnp.int32),
        pltpu.VMEM((CHUNK, DIM), jnp.float32),
        pltpu.VMEM((CHUNK, DIM), jnp.float32),
        pltpu.SemaphoreType.DMA,
        pltpu.SemaphoreType.DMA,
    ],
)


@jax.jit
def kernel(x, table):
    xf = x.reshape(B).astype(jnp.int32)
    out = _embed_call(xf, table)
    return out.reshape(ROWS, COLS, DIM)
